# baseline (speedup 1.0000x reference)
import functools

import jax
import jax.numpy as jnp
from jax import lax
from jax.experimental import pallas as pl
from jax.experimental.pallas import tpu as pltpu
from jax.experimental.pallas import tpu_sc as plsc

NC = 2
NS = 16
NW = NC * NS
LANES = 16

WDIM = 64
TDIM = 32
NTAGS = 50

SPG = 2
BLKS = (128, 72)


def _tag_ids(iv):
    f = iv.astype(jnp.float32) * jnp.float32(1.0 / NTAGS)
    q = f.astype(jnp.int32)
    r = iv - q * jnp.int32(NTAGS)
    return jnp.where(r >= NTAGS, r - jnp.int32(NTAGS), r)


def _body(T, L, idx_hbm, wtab_hbm, ttab_hbm, out_hbm,
          idx_v, tag_v, wbuf, tbuf,
          isem0, isem1, gsem0, gsem1, osem0, osem1):
    isem = (isem0, isem1)
    gsem = (gsem0, gsem1)
    osem = (osem0, osem1)
    grp = SPG * L
    wid = lax.axis_index("s") * NC + lax.axis_index("c")
    s_base = wid * (2 * T * SPG)
    t_base = s_base * L

    def out_slices(s0):
        return (out_hbm.at[pl.ds(s0, SPG), :, pl.ds(0, WDIM)],
                out_hbm.at[pl.ds(s0, SPG), :, pl.ds(WDIM, TDIM)])

    for p in (0, 1):
        pltpu.async_copy(idx_hbm.at[pl.ds(t_base + p * grp, grp)],
                         idx_v.at[p], isem[p])

    def dbl(t, carry):
        gcopies = [[], []]
        for p in (0, 1):
            g = 2 * t + p
            tok0 = t_base + g * grp
            s0 = s_base + g * SPG

            ows, ots = out_slices(s0)

            @pl.when(t > 0)
            def _():
                pltpu.make_async_copy(wbuf.at[p], ows, osem[p]).wait()
                pltpu.make_async_copy(tbuf.at[p], ots, osem[p]).wait()

            pltpu.make_async_copy(idx_hbm.at[pl.ds(tok0, grp)],
                                  idx_v.at[p], isem[p]).wait()
            for c in range(grp // LANES):
                sl = pl.ds(c * LANES, LANES)
                tag_v[p, sl] = _tag_ids(idx_v[p, sl])

            for sent in range(SPG):
                off = 0
                for blk in BLKS:
                    isl = pl.ds(sent * L + off, blk)
                    gcopies[p].append(pltpu.async_copy(
                        wtab_hbm.at[idx_v.at[p, isl]],
                        wbuf.at[p, sent, pl.ds(off, blk)], gsem[p]))
                    gcopies[p].append(pltpu.async_copy(
                        ttab_hbm.at[tag_v.at[p, isl]],
                        tbuf.at[p, sent, pl.ds(off, blk)], gsem[p]))
                    off += blk

        for p in (0, 1):
            g = 2 * t + p
            tok0 = t_base + g * grp
            s0 = s_base + g * SPG

            for cp in gcopies[p]:
                cp.wait()

            @pl.when(t < T - 1)
            def _():
                pltpu.async_copy(
                    idx_hbm.at[pl.ds(tok0 + 2 * grp, grp)],
                    idx_v.at[p], isem[p])

            ows, ots = out_slices(s0)
            pltpu.async_copy(wbuf.at[p], ows, osem[p])
            pltpu.async_copy(tbuf.at[p], ots, osem[p])
        return carry

    lax.fori_loop(0, T, dbl, 0)

    for p in (0, 1):
        s0 = s_base + (2 * (T - 1) + p) * SPG
        ows, ots = out_slices(s0)
        pltpu.make_async_copy(wbuf.at[p], ows, osem[p]).wait()
        pltpu.make_async_copy(tbuf.at[p], ots, osem[p]).wait()


def kernel(sentence, word_table, tag_table):
    B, L = sentence.shape
    n = B * L
    grp = SPG * L
    assert sum(BLKS) == L and grp % LANES == 0 and (B // NW) % (2 * SPG) == 0
    T = B // (NW * 2 * SPG)

    idx = sentence.reshape(n).astype(jnp.int32)

    mesh = plsc.VectorSubcoreMesh(core_axis_name="c", subcore_axis_name="s")
    return pl.kernel(
        functools.partial(_body, T, L),
        out_type=jax.ShapeDtypeStruct((B, L, WDIM + TDIM), jnp.float32),
        mesh=mesh,
        compiler_params=pltpu.CompilerParams(use_tc_tiling_on_sc=False),
        scratch_types=[
            pltpu.VMEM((2, grp), jnp.int32),
            pltpu.VMEM((2, grp), jnp.int32),
            pltpu.VMEM((2, SPG, L, WDIM), jnp.float32),
            pltpu.VMEM((2, SPG, L, TDIM), jnp.float32),
        ] + [pltpu.SemaphoreType.DMA] * 6,
    )(idx, word_table, tag_table)


# speedup vs baseline: 1.0013x; 1.0013x over previous
import functools

import jax
import jax.numpy as jnp
from jax import lax
from jax.experimental import pallas as pl
from jax.experimental.pallas import tpu as pltpu
from jax.experimental.pallas import tpu_sc as plsc

NC = 2
NS = 16
NW = NC * NS
LANES = 16

WDIM = 64
TDIM = 32
NTAGS = 50

GRP = 512


def _tag_ids(iv):
    f = iv.astype(jnp.float32) * jnp.float32(1.0 / NTAGS)
    q = f.astype(jnp.int32)
    r = iv - q * jnp.int32(NTAGS)
    return jnp.where(r >= NTAGS, r - jnp.int32(NTAGS), r)


def _body(T, idx_hbm, wtab_hbm, ttab_hbm, out_hbm,
          idx_v, tag_v, wbuf, tbuf,
          isem0, isem1, gsem0, gsem1, osem0, osem1):
    isem = (isem0, isem1)
    gsem = (gsem0, gsem1)
    osem = (osem0, osem1)
    wid = lax.axis_index("s") * NC + lax.axis_index("c")
    t_base = wid * (2 * T * GRP)

    def out_slices(tok0):
        return (out_hbm.at[pl.ds(tok0, GRP), pl.ds(0, WDIM)],
                out_hbm.at[pl.ds(tok0, GRP), pl.ds(WDIM, TDIM)])

    for p in (0, 1):
        pltpu.async_copy(idx_hbm.at[pl.ds(t_base + p * GRP, GRP)],
                         idx_v.at[p], isem[p])

    def dbl(t, carry):
        gcopies = [[], []]
        for p in (0, 1):
            tok0 = t_base + (2 * t + p) * GRP

            ows, ots = out_slices(tok0)

            @pl.when(t > 0)
            def _():
                pltpu.make_async_copy(wbuf.at[p], ows, osem[p]).wait()
                pltpu.make_async_copy(tbuf.at[p], ots, osem[p]).wait()

            pltpu.make_async_copy(idx_hbm.at[pl.ds(tok0, GRP)],
                                  idx_v.at[p], isem[p]).wait()
            for c in range(GRP // LANES):
                sl = pl.ds(c * LANES, LANES)
                tag_v[p, sl] = _tag_ids(idx_v[p, sl])

            gcopies[p].append(pltpu.async_copy(
                wtab_hbm.at[idx_v.at[p]], wbuf.at[p], gsem[p]))
            gcopies[p].append(pltpu.async_copy(
                ttab_hbm.at[tag_v.at[p]], tbuf.at[p], gsem[p]))

        for p in (0, 1):
            tok0 = t_base + (2 * t + p) * GRP

            for cp in gcopies[p]:
                cp.wait()

            @pl.when(t < T - 1)
            def _():
                pltpu.async_copy(idx_hbm.at[pl.ds(tok0 + 2 * GRP, GRP)],
                                 idx_v.at[p], isem[p])

            ows, ots = out_slices(tok0)
            pltpu.async_copy(wbuf.at[p], ows, osem[p])
            pltpu.async_copy(tbuf.at[p], ots, osem[p])
        return carry

    lax.fori_loop(0, T, dbl, 0)

    for p in (0, 1):
        tok0 = t_base + (2 * (T - 1) + p) * GRP
        ows, ots = out_slices(tok0)
        pltpu.make_async_copy(wbuf.at[p], ows, osem[p]).wait()
        pltpu.make_async_copy(tbuf.at[p], ots, osem[p]).wait()


def kernel(sentence, word_table, tag_table):
    B, L = sentence.shape
    n = B * L
    assert n % (NW * 2 * GRP) == 0
    T = n // (NW * 2 * GRP)

    idx = sentence.reshape(n).astype(jnp.int32)

    mesh = plsc.VectorSubcoreMesh(core_axis_name="c", subcore_axis_name="s")
    out = pl.kernel(
        functools.partial(_body, T),
        out_type=jax.ShapeDtypeStruct((n, WDIM + TDIM), jnp.float32),
        mesh=mesh,
        compiler_params=pltpu.CompilerParams(use_tc_tiling_on_sc=False),
        scratch_types=[
            pltpu.VMEM((2, GRP), jnp.int32),
            pltpu.VMEM((2, GRP), jnp.int32),
            pltpu.VMEM((2, GRP, WDIM), jnp.float32),
            pltpu.VMEM((2, GRP, TDIM), jnp.float32),
        ] + [pltpu.SemaphoreType.DMA] * 6,
    )(idx, word_table, tag_table)
    return out.reshape(B, L, WDIM + TDIM)
